# Initial kernel scaffold; baseline (speedup 1.0000x reference)
import jax
import jax.numpy as jnp
from jax.experimental import pallas as pl


def kernel(x, edge_index, query_node_indices, W1, b1, W2, b2, Wm1, bm1, Wm2, bm2):
    raise NotImplementedError("write your pallas kernel here")


# speedup vs baseline: 18.2991x; 18.2991x over previous
import functools

import jax
import jax.numpy as jnp
from jax import lax
from jax.experimental import pallas as pl
from jax.experimental.pallas import tpu as pltpu
from jax.experimental.pallas import tpu_sc as plsc

N = 10000
E = 320000
D = 128
H = 128
O = 32
Q = 1024
MLP_H = 64

NPAD = 10240
NC = 2
NS = 16
CHUNK = 128
E_PER_SC = E // NC
NCH_SC = E_PER_SC // CHUNK
NCH_BASE = NCH_SC // NS
NCH_REM = NCH_SC % NS
ROWS_PER_TILE = NPAD // NS
ZROWS = 128

_MESH = dict(core_axis_name="c", subcore_axis_name="s", num_cores=NC,
             num_subcores=NS)
_SC_PARAMS = pltpu.CompilerParams(use_tc_tiling_on_sc=False)


def _fill_zeros_2d(ref, nrows, ncols):
    zv = jnp.zeros((16,), jnp.float32)

    def body(r, _):
        for j in range(ncols // 16):
            ref[r, pl.ds(j * 16, 16)] = zv
        return 0

    lax.fori_loop(0, nrows, body, 0)


def _make_segsum(F):
    mesh = plsc.VectorSubcoreMesh(**_MESH)

    @functools.partial(
        pl.kernel,
        out_type=jax.ShapeDtypeStruct((NC, NPAD, F), jnp.float32),
        mesh=mesh,
        compiler_params=_SC_PARAMS,
        scratch_types=[
            pltpu.VMEM((CHUNK,), jnp.int32),
            pltpu.VMEM((CHUNK,), jnp.int32),
            pltpu.VMEM((CHUNK, F), jnp.float32),
            pltpu.VMEM((ZROWS, F), jnp.float32),
            pltpu.VMEM_SHARED((NPAD, F), jnp.float32),
            pltpu.SemaphoreType.DMA,
        ],
    )
    def segsum(hs, src, dst, out, src_v, dst_v, rows_v, zer_v, acc, sem):
        c = lax.axis_index("c")
        s = lax.axis_index("s")
        _fill_zeros_2d(zer_v, ZROWS, F)
        for t in range(ROWS_PER_TILE // ZROWS):
            pltpu.sync_copy(zer_v, acc.at[pl.ds(s * ROWS_PER_TILE + t * ZROWS, ZROWS)])
        plsc.subcore_barrier()

        n_i = jnp.where(s < NCH_REM, NCH_BASE + 1, NCH_BASE)

        def body(i, _):
            base = c * E_PER_SC + (i * NS + s) * CHUNK
            pltpu.sync_copy(src.at[pl.ds(base, CHUNK)], src_v)
            pltpu.sync_copy(dst.at[pl.ds(base, CHUNK)], dst_v)
            pltpu.async_copy(hs.at[src_v], rows_v, sem).wait()
            pltpu.sync_copy(rows_v, acc.at[dst_v], add=True)
            return 0

        lax.fori_loop(0, n_i, body, 0)
        plsc.subcore_barrier()
        pltpu.sync_copy(acc.at[pl.ds(s * ROWS_PER_TILE, ROWS_PER_TILE)],
                        out.at[c, pl.ds(s * ROWS_PER_TILE, ROWS_PER_TILE), :])

    return segsum


_segsum_h = _make_segsum(H)
_segsum_o = _make_segsum(O)


@functools.partial(
    pl.kernel,
    out_type=jax.ShapeDtypeStruct((NC, NPAD), jnp.float32),
    mesh=plsc.VectorSubcoreMesh(**_MESH),
    compiler_params=_SC_PARAMS,
    scratch_types=[
        pltpu.VMEM((CHUNK,), jnp.int32),
        pltpu.VMEM((CHUNK,), jnp.float32),
        pltpu.VMEM((ROWS_PER_TILE,), jnp.float32),
        pltpu.VMEM_SHARED((NPAD,), jnp.float32),
    ],
)
def _degree(dst, out, dst_v, ones_v, zer_v, acc):
    c = lax.axis_index("c")
    s = lax.axis_index("s")
    one = jnp.full((16,), 1.0, jnp.float32)
    zero = jnp.zeros((16,), jnp.float32)
    for j in range(CHUNK // 16):
        ones_v[pl.ds(j * 16, 16)] = one

    def zbody(r, _):
        zer_v[pl.ds(r * 16, 16)] = zero
        return 0

    lax.fori_loop(0, ROWS_PER_TILE // 16, zbody, 0)
    pltpu.sync_copy(zer_v, acc.at[pl.ds(s * ROWS_PER_TILE, ROWS_PER_TILE)])
    plsc.subcore_barrier()

    n_i = jnp.where(s < NCH_REM, NCH_BASE + 1, NCH_BASE)

    def body(i, _):
        base = c * E_PER_SC + (i * NS + s) * CHUNK
        pltpu.sync_copy(dst.at[pl.ds(base, CHUNK)], dst_v)
        pltpu.sync_copy(ones_v, acc.at[dst_v], add=True)
        return 0

    lax.fori_loop(0, n_i, body, 0)
    plsc.subcore_barrier()
    pltpu.sync_copy(acc.at[pl.ds(s * ROWS_PER_TILE, ROWS_PER_TILE)],
                    out.at[c, pl.ds(s * ROWS_PER_TILE, ROWS_PER_TILE)])


_Q_PER_TILE = Q // (NC * NS)


@functools.partial(
    pl.kernel,
    out_type=jax.ShapeDtypeStruct((Q, O), jnp.float32),
    mesh=plsc.VectorSubcoreMesh(**_MESH),
    compiler_params=_SC_PARAMS,
    scratch_types=[
        pltpu.VMEM((_Q_PER_TILE,), jnp.int32),
        pltpu.VMEM((_Q_PER_TILE, O), jnp.float32),
        pltpu.SemaphoreType.DMA,
    ],
)
def _gather_queries(table, qidx, out, idx_v, rows_v, sem):
    c = lax.axis_index("c")
    s = lax.axis_index("s")
    w = s * NC + c
    base = w * _Q_PER_TILE
    pltpu.sync_copy(qidx.at[pl.ds(base, _Q_PER_TILE)], idx_v)
    pltpu.async_copy(table.at[idx_v], rows_v, sem).wait()
    pltpu.sync_copy(rows_v, out.at[pl.ds(base, _Q_PER_TILE), :])


_NBLK = NPAD // 1024


def _tc_scale1(x_r, w_r, dp_r, hs_r, dinv_r):
    deg = dp_r[0] + dp_r[1] + 1.0
    dinv = lax.rsqrt(jnp.maximum(deg, 1e-12))
    hs_r[...] = jnp.dot(x_r[...], w_r[...],
                        preferred_element_type=jnp.float32) * dinv
    dinv_r[...] = dinv


_stage1 = pl.pallas_call(
    _tc_scale1,
    grid=(_NBLK,),
    in_specs=[
        pl.BlockSpec((1024, D), lambda i: (i, 0)),
        pl.BlockSpec((D, H), lambda i: (0, 0)),
        pl.BlockSpec((NC, 1024, 1), lambda i: (0, i, 0)),
    ],
    out_specs=[
        pl.BlockSpec((1024, H), lambda i: (i, 0)),
        pl.BlockSpec((1024, 1), lambda i: (i, 0)),
    ],
    out_shape=[
        jax.ShapeDtypeStruct((NPAD, H), jnp.float32),
        jax.ShapeDtypeStruct((NPAD, 1), jnp.float32),
    ],
)


def _tc_combine1(p_r, hs_r, dinv_r, b1_r, w2_r, h2s_r):
    seg = p_r[0] + p_r[1] + hs_r[...]
    out1 = jnp.maximum(dinv_r[...] * seg + b1_r[...], 0.0)
    h2s_r[...] = jnp.dot(out1, w2_r[...],
                         preferred_element_type=jnp.float32) * dinv_r[...]


_stage2 = pl.pallas_call(
    _tc_combine1,
    grid=(_NBLK,),
    in_specs=[
        pl.BlockSpec((NC, 1024, H), lambda i: (0, i, 0)),
        pl.BlockSpec((1024, H), lambda i: (i, 0)),
        pl.BlockSpec((1024, 1), lambda i: (i, 0)),
        pl.BlockSpec((1, H), lambda i: (0, 0)),
        pl.BlockSpec((H, O), lambda i: (0, 0)),
    ],
    out_specs=pl.BlockSpec((1024, O), lambda i: (i, 0)),
    out_shape=jax.ShapeDtypeStruct((NPAD, O), jnp.float32),
)


def _tc_combine2(q_r, h2s_r, dinv_r, b2_r, out_r):
    seg = q_r[0] + q_r[1] + h2s_r[...]
    out_r[...] = jnp.maximum(dinv_r[...] * seg + b2_r[...], 0.0)


_stage3 = pl.pallas_call(
    _tc_combine2,
    grid=(_NBLK,),
    in_specs=[
        pl.BlockSpec((NC, 1024, O), lambda i: (0, i, 0)),
        pl.BlockSpec((1024, O), lambda i: (i, 0)),
        pl.BlockSpec((1024, 1), lambda i: (i, 0)),
        pl.BlockSpec((1, O), lambda i: (0, 0)),
    ],
    out_specs=pl.BlockSpec((1024, O), lambda i: (i, 0)),
    out_shape=jax.ShapeDtypeStruct((NPAD, O), jnp.float32),
)


def _tc_mlp(q_r, wm1_r, bm1_r, wm2_r, bm2_r, out_r):
    z = jnp.maximum(jnp.dot(q_r[...], wm1_r[...],
                            preferred_element_type=jnp.float32) + bm1_r[...], 0.0)
    out_r[...] = jnp.dot(z, wm2_r[...],
                         preferred_element_type=jnp.float32) + bm2_r[...]


_mlp = pl.pallas_call(
    _tc_mlp,
    out_shape=jax.ShapeDtypeStruct((Q, 1), jnp.float32),
)


@jax.jit
def kernel(x, edge_index, query_node_indices, W1, b1, W2, b2, Wm1, bm1, Wm2, bm2):
    src = edge_index[0]
    dst = edge_index[1]
    x_pad = jnp.zeros((NPAD, D), jnp.float32).at[:N].set(x)

    degp = _degree(dst)
    hs, dinv = _stage1(x_pad, W1, degp.reshape(NC, NPAD, 1))
    p1 = _segsum_h(hs, src, dst)
    h2s = _stage2(p1, hs, dinv, b1.reshape(1, H), W2)
    p2 = _segsum_o(h2s, src, dst)
    out2 = _stage3(p2, h2s, dinv, b2.reshape(1, O))
    qrows = _gather_queries(out2, query_node_indices)
    logits = _mlp(qrows, Wm1, bm1.reshape(1, MLP_H), Wm2, bm2.reshape(1, 1))
    return logits[:, 0]
